# baseline (speedup 1.0000x reference)
import jax
import jax.numpy as jnp
from jax import lax
from jax.experimental import pallas as pl
from jax.experimental.pallas import tpu as pltpu
from jax.experimental.pallas import tpu_sc as plsc

NC, NS, LANES = 2, 16, 16
NW = NC * NS
NB = 4096
NT = 50
D = 64
CH = 128
NCH = NT
R = 5
MAP_AHEAD = 3


def _body(text_hbm, tw_hbm, fw_hbm, tmap_hbm, fmap_hbm, out_hbm,
          tok_v, tidx_v, fidx_v, fw_v, rows_a, tr_b,
          sem_row, sem_out, sem_map):
    wid = lax.axis_index("s") * NC + lax.axis_index("c")

    pltpu.sync_copy(fw_hbm, fw_v)
    pltpu.sync_copy(text_hbm.at[:, pl.ds(wid * CH, CH)], tok_v)

    def map_copies(c, s):
        return (
            pltpu.make_async_copy(tmap_hbm.at[tok_v.at[c]], tidx_v.at[c],
                                  sem_map.at[s]),
            pltpu.make_async_copy(fmap_hbm.at[tok_v.at[c]], fidx_v.at[c],
                                  sem_map.at[s]),
        )

    def row_copies(c, s):
        return (
            pltpu.make_async_copy(tw_hbm.at[tidx_v.at[c]], rows_a[s],
                                  sem_row.at[s]),
        )

    def out_copy(c, s):
        return pltpu.make_async_copy(
            tr_b[s], out_hbm.at[c, :, wid], sem_out.at[s])

    def issue(copies):
        for cp in copies:
            cp.start()

    def drain(copies):
        for cp in copies:
            cp.wait()

    def add_frozen(c, s):
        @pl.loop(0, CH // LANES)
        def _(g):
            fvec = fidx_v[c, pl.ds(g * LANES, LANES)]
            nfrozen = plsc.all_reduce_population_count(fvec != 0)

            @pl.when(nfrozen[0] > 0)
            def _():
                base = g * LANES
                for k in range(LANES):
                    f = fvec[k]

                    @pl.when(f != 0)
                    def _():
                        for cc in range(D // LANES):
                            sl = pl.ds(cc * LANES, LANES)
                            rows_a[s][base + k, sl] = (
                                rows_a[s][base + k, sl] + fw_v[f, sl])

    def transpose(s):
        @pl.loop(0, D // 8)
        def _(dt):
            for bg in range(CH // LANES):
                rid = lax.iota(jnp.int32, LANES) + bg * LANES
                for ds_ in range(8):
                    d = dt * 8 + ds_
                    col = jnp.zeros((LANES,), jnp.int32) + d
                    v = plsc.load_gather(rows_a[s], [rid, col])
                    tr_b[s][dt, ds_, pl.ds(bg * LANES, LANES)] = v

    for c in range(MAP_AHEAD):
        issue(map_copies(c, c % R))
    drain(map_copies(0, 0))
    issue(row_copies(0, 0))

    @pl.loop(0, NCH, step=R)
    def _(i0):
        for b in range(R):
            i = i0 + b
            drain(row_copies(i, b))
            add_frozen(i, b)

            @pl.when(i >= R)
            def _():
                out_copy(i - R, b).wait()

            transpose(b)
            out_copy(i, b).start()

            j = i + 1
            bj = (b + 1) % R

            @pl.when(j < NCH)
            def _():
                drain(map_copies(j, bj))
                issue(row_copies(j, bj))

            m = i + MAP_AHEAD
            bm = (b + MAP_AHEAD) % R

            @pl.when(m < NCH)
            def _():
                issue(map_copies(m, bm))

    for b in range(R):
        out_copy(NCH - R + b, b).wait()


_run = pl.kernel(
    _body,
    out_type=jax.ShapeDtypeStruct((NT, 8, NB // CH, 8, CH), jnp.float32),
    mesh=plsc.VectorSubcoreMesh(core_axis_name="c", subcore_axis_name="s"),
    compiler_params=pltpu.CompilerParams(use_tc_tiling_on_sc=False,
                                         needs_layout_passes=False),
    scratch_types=[
        pltpu.VMEM((NCH, CH), jnp.int32),
        pltpu.VMEM((NCH, CH), jnp.int32),
        pltpu.VMEM((NCH, CH), jnp.int32),
        pltpu.VMEM((65, D), jnp.float32),
        [pltpu.VMEM((CH, D), jnp.float32) for _ in range(R)],
        [pltpu.VMEM((8, 8, CH), jnp.float32) for _ in range(R)],
        pltpu.SemaphoreType.DMA((R,)),
        pltpu.SemaphoreType.DMA((R,)),
        pltpu.SemaphoreType.DMA((R,)),
    ],
)


@jax.jit
def kernel(text_input, trainable_weight, frozen_weight, trainable_map,
           frozen_map):
    out5d = _run(text_input.T, trainable_weight, frozen_weight,
                 trainable_map, frozen_map)
    return out5d.transpose(2, 4, 0, 1, 3).reshape(NB, NT, D)


# speedup vs baseline: 1.2458x; 1.2458x over previous
import jax
import jax.numpy as jnp
from jax import lax
from jax.experimental import pallas as pl
from jax.experimental.pallas import tpu as pltpu
from jax.experimental.pallas import tpu_sc as plsc

NC, NS, LANES = 2, 16, 16
NW = NC * NS
NB = 4096
NT = 50
D = 64
CH = 128
NCH = NT
R = 5
MAP_AHEAD = 3


def _body(text_hbm, tw_hbm, fw_hbm, tmap_hbm, fmap_hbm, out_hbm,
          tok_v, tidx_v, fidx_v, fw_v, rows_a, tr_b,
          sem_row, sem_out, sem_map):
    wid = lax.axis_index("s") * NC + lax.axis_index("c")

    pltpu.sync_copy(fw_hbm, fw_v)
    pltpu.sync_copy(text_hbm.at[:, pl.ds(wid * CH, CH)], tok_v)

    def map_copies(c, s):
        return (
            pltpu.make_async_copy(tmap_hbm.at[tok_v.at[c]], tidx_v.at[c],
                                  sem_map.at[s]),
            pltpu.make_async_copy(fmap_hbm.at[tok_v.at[c]], fidx_v.at[c],
                                  sem_map.at[s]),
        )

    def row_copies(c, s):
        return (
            pltpu.make_async_copy(tw_hbm.at[tidx_v.at[c]], rows_a[s],
                                  sem_row.at[s]),
        )

    def out_copy(c, s):
        return pltpu.make_async_copy(
            tr_b[s], out_hbm.at[c, :, wid], sem_out.at[s])

    def issue(copies):
        for cp in copies:
            cp.start()

    def drain(copies):
        for cp in copies:
            cp.wait()

    def add_frozen(c, s):
        @pl.loop(0, CH // LANES)
        def _(g):
            fvec = fidx_v[c, pl.ds(g * LANES, LANES)]
            nfrozen = plsc.all_reduce_population_count(fvec != 0)

            @pl.when(nfrozen[0] > 0)
            def _():
                base = g * LANES
                for k in range(LANES):
                    f = fvec[k]

                    @pl.when(f != 0)
                    def _():
                        for cc in range(D // LANES):
                            sl = pl.ds(cc * LANES, LANES)
                            rows_a[s][base + k, sl] = (
                                rows_a[s][base + k, sl] + fw_v[f, sl])

    def transpose(s):
        zeros = jnp.zeros((LANES,), jnp.int32)

        @plsc.parallel_loop(0, D // 8, unroll=2)
        def _(dt):
            for bg in range(CH // LANES):
                rid = lax.iota(jnp.int32, LANES) + bg * LANES
                for ds_ in range(8):
                    d = dt * 8 + ds_
                    v = plsc.load_gather(rows_a[s], [rid, zeros + d])
                    tr_b[s][dt, ds_, pl.ds(bg * LANES, LANES)] = v

    for c in range(MAP_AHEAD):
        issue(map_copies(c, c % R))
    drain(map_copies(0, 0))
    issue(row_copies(0, 0))

    @pl.loop(0, NCH, step=R)
    def _(i0):
        for b in range(R):
            i = i0 + b
            drain(row_copies(i, b))
            add_frozen(i, b)

            @pl.when(i >= R)
            def _():
                out_copy(i - R, b).wait()

            transpose(b)
            out_copy(i, b).start()

            j = i + 1
            bj = (b + 1) % R

            @pl.when(j < NCH)
            def _():
                drain(map_copies(j, bj))
                issue(row_copies(j, bj))

            m = i + MAP_AHEAD
            bm = (b + MAP_AHEAD) % R

            @pl.when(m < NCH)
            def _():
                issue(map_copies(m, bm))

    for b in range(R):
        out_copy(NCH - R + b, b).wait()


_run = pl.kernel(
    _body,
    out_type=jax.ShapeDtypeStruct((NT, 8, NB // CH, 8, CH), jnp.float32),
    mesh=plsc.VectorSubcoreMesh(core_axis_name="c", subcore_axis_name="s"),
    compiler_params=pltpu.CompilerParams(use_tc_tiling_on_sc=False,
                                         needs_layout_passes=False),
    scratch_types=[
        pltpu.VMEM((NCH, CH), jnp.int32),
        pltpu.VMEM((NCH, CH), jnp.int32),
        pltpu.VMEM((NCH, CH), jnp.int32),
        pltpu.VMEM((65, D), jnp.float32),
        [pltpu.VMEM((CH, D), jnp.float32) for _ in range(R)],
        [pltpu.VMEM((8, 8, CH), jnp.float32) for _ in range(R)],
        pltpu.SemaphoreType.DMA((R,)),
        pltpu.SemaphoreType.DMA((R,)),
        pltpu.SemaphoreType.DMA((R,)),
    ],
)


@jax.jit
def kernel(text_input, trainable_weight, frozen_weight, trainable_map,
           frozen_map):
    out5d = _run(text_input.T, trainable_weight, frozen_weight,
                 trainable_map, frozen_map)
    return out5d.transpose(2, 4, 0, 1, 3).reshape(NB, NT, D)


# speedup vs baseline: 1.3238x; 1.0626x over previous
import jax
import jax.numpy as jnp
from jax import lax
from jax.experimental import pallas as pl
from jax.experimental.pallas import tpu as pltpu
from jax.experimental.pallas import tpu_sc as plsc

NC, NS, LANES = 2, 16, 16
NW = NC * NS
NB = 4096
NT = 50
D = 64
CH = 128
NCH = NT
R = 5
MAP_AHEAD = 3


def _body(text_hbm, tw_hbm, fw_hbm, tmap_hbm, fmap_hbm, out_hbm,
          tok_v, tidx_v, fidx_v, fw_v, rows_a, tr_b,
          sem_row, sem_out, sem_map):
    wid = lax.axis_index("s") * NC + lax.axis_index("c")

    pltpu.sync_copy(fw_hbm, fw_v)
    pltpu.sync_copy(text_hbm.at[:, pl.ds(wid * CH, CH)], tok_v)

    def map_copies(c, s):
        return (
            pltpu.make_async_copy(tmap_hbm.at[tok_v.at[c]], tidx_v.at[c],
                                  sem_map.at[s]),
            pltpu.make_async_copy(fmap_hbm.at[tok_v.at[c]], fidx_v.at[c],
                                  sem_map.at[s]),
        )

    def row_copies(c, s):
        return (
            pltpu.make_async_copy(tw_hbm.at[tidx_v.at[c]], rows_a[s],
                                  sem_row.at[s]),
        )

    def out_copy(c, s):
        return pltpu.make_async_copy(
            tr_b[s], out_hbm.at[c, :, wid], sem_out.at[s])

    def issue(copies):
        for cp in copies:
            cp.start()

    def drain(copies):
        for cp in copies:
            cp.wait()

    def add_frozen(c, s):
        @pl.loop(0, CH // LANES)
        def _(g):
            fvec = fidx_v[c, pl.ds(g * LANES, LANES)]
            nfrozen = plsc.all_reduce_population_count(fvec != 0)

            @pl.when(nfrozen[0] > 0)
            def _():
                base = g * LANES
                for k in range(LANES):
                    f = fvec[k]

                    @pl.when(f != 0)
                    def _():
                        for cc in range(D // LANES):
                            sl = pl.ds(cc * LANES, LANES)
                            rows_a[s][base + k, sl] = (
                                rows_a[s][base + k, sl] + fw_v[f, sl])

    def transpose(s):
        zeros = jnp.zeros((LANES,), jnp.int32)

        @plsc.parallel_loop(0, D // 8, unroll=2)
        def _(dt):
            for bg in range(CH // LANES):
                rid = lax.iota(jnp.int32, LANES) + bg * LANES
                vs = [plsc.load_gather(rows_a[s], [rid, zeros + (dt * 8 + ds_)])
                      for ds_ in range(8)]
                for ds_ in range(8):
                    tr_b[s][dt, ds_, pl.ds(bg * LANES, LANES)] = vs[ds_]

    for c in range(MAP_AHEAD):
        issue(map_copies(c, c % R))
    drain(map_copies(0, 0))
    issue(row_copies(0, 0))

    @pl.loop(0, NCH, step=R)
    def _(i0):
        for b in range(R):
            i = i0 + b
            drain(row_copies(i, b))
            add_frozen(i, b)

            @pl.when(i >= R)
            def _():
                out_copy(i - R, b).wait()

            transpose(b)
            out_copy(i, b).start()

            j = i + 1
            bj = (b + 1) % R

            @pl.when(j < NCH)
            def _():
                drain(map_copies(j, bj))
                issue(row_copies(j, bj))

            m = i + MAP_AHEAD
            bm = (b + MAP_AHEAD) % R

            @pl.when(m < NCH)
            def _():
                issue(map_copies(m, bm))

    for b in range(R):
        out_copy(NCH - R + b, b).wait()


_run = pl.kernel(
    _body,
    out_type=jax.ShapeDtypeStruct((NT, 8, NB // CH, 8, CH), jnp.float32),
    mesh=plsc.VectorSubcoreMesh(core_axis_name="c", subcore_axis_name="s"),
    compiler_params=pltpu.CompilerParams(use_tc_tiling_on_sc=False,
                                         needs_layout_passes=False),
    scratch_types=[
        pltpu.VMEM((NCH, CH), jnp.int32),
        pltpu.VMEM((NCH, CH), jnp.int32),
        pltpu.VMEM((NCH, CH), jnp.int32),
        pltpu.VMEM((65, D), jnp.float32),
        [pltpu.VMEM((CH, D), jnp.float32) for _ in range(R)],
        [pltpu.VMEM((8, 8, CH), jnp.float32) for _ in range(R)],
        pltpu.SemaphoreType.DMA((R,)),
        pltpu.SemaphoreType.DMA((R,)),
        pltpu.SemaphoreType.DMA((R,)),
    ],
)


@jax.jit
def kernel(text_input, trainable_weight, frozen_weight, trainable_map,
           frozen_map):
    out5d = _run(text_input.T, trainable_weight, frozen_weight,
                 trainable_map, frozen_map)
    return out5d.transpose(2, 4, 0, 1, 3).reshape(NB, NT, D)
